# baseline (speedup 1.0000x reference)
import functools

import jax
import jax.numpy as jnp
from jax import lax
from jax.experimental import pallas as pl
from jax.experimental.pallas import tpu as pltpu
from jax.experimental.pallas import tpu_sc as plsc

NC = 2
NS = 16
L = 16
NW = NC * NS

BATCH = 16384
DIM = 128
ROWS_PER_W = BATCH // NW
NSEG = 2
SEG = ROWS_PER_W // NSEG

_MAGIC = 12582912.0


@functools.partial(
    pl.kernel,
    out_type=jax.ShapeDtypeStruct((BATCH, DIM), jnp.float32),
    mesh=plsc.VectorSubcoreMesh(core_axis_name="c", subcore_axis_name="s"),
    scratch_types=[
        pltpu.VMEM((3, DIM), jnp.float32),
        pltpu.VMEM((1, DIM), jnp.float32),
        pltpu.VMEM((52, DIM), jnp.float32),
        pltpu.VMEM((53, DIM), jnp.float32),
        pltpu.VMEM((ROWS_PER_W,), jnp.float32),
        pltpu.VMEM((ROWS_PER_W, DIM), jnp.float32),
        pltpu.SemaphoreType.DMA,
        pltpu.SemaphoreType.DMA,
        pltpu.SemaphoreType.DMA,
    ],
    compiler_params=pltpu.CompilerParams(
        needs_layout_passes=False, use_tc_tiling_on_sc=False
    ),
)
def _embed(cards_hbm, rank_hbm, suit_hbm, card_hbm, out_hbm,
           rank_v, suit_v, card_v, comb_v, cards_v, rows_v,
           sem, cards_sem, out_sem):
    wid = lax.axis_index("s") * NC + lax.axis_index("c")
    base = wid * ROWS_PER_W

    cards_cp = pltpu.async_copy(
        cards_hbm.at[pl.ds(base, ROWS_PER_W)], cards_v, cards_sem)
    loads = [
        pltpu.async_copy(rank_hbm, rank_v, sem),
        pltpu.async_copy(suit_hbm, suit_v, sem),
        pltpu.async_copy(card_hbm, card_v, sem),
    ]
    for cp in loads:
        cp.wait()

    for d in range(DIM // L):
        ds = pl.ds(d * L, L)
        suit_row = suit_v[0, ds]
        for j in range(3):
            rank_v[j, ds] = rank_v[j, ds] + suit_row
        comb_v[52, ds] = suit_row
    for k in range(52):
        rk = min(k // 4, 2)
        for d in range(DIM // L):
            ds = pl.ds(d * L, L)
            comb_v[k, ds] = card_v[k, ds] + rank_v[rk, ds]

    def row_loads(cj):
        return [comb_v[cj, pl.ds(d * L, L)] for d in range(DIM // L)]

    def row_stores(row, vals):
        for d in range(DIM // L):
            rows_v[row, pl.ds(d * L, L)] = vals[d]

    def expand(lo, hi):
        @plsc.parallel_loop(lo, hi, unroll=2)
        def _(g):
            cards_f = cards_v[pl.ds(g * L, L)]
            rounded = (cards_f + _MAGIC) - _MAGIC
            c = rounded.astype(jnp.int32)
            idx = jnp.where(c < 0, 52, jnp.minimum(c, 51))
            cjs = [idx[j] for j in range(L)]
            vals = row_loads(cjs[0])
            for j in range(1, L):
                nxt = []
                for d in range(DIM // L):
                    nxt.append(comb_v[cjs[j], pl.ds(d * L, L)])
                    rows_v[g * L + (j - 1), pl.ds(d * L, L)] = vals[d]
                vals = nxt
            row_stores(g * L + (L - 1), vals)

    cards_cp.wait()
    out_cps = []
    for s in range(NSEG):
        expand(s * (SEG // L), (s + 1) * (SEG // L))
        out_cps.append(
            pltpu.async_copy(rows_v.at[pl.ds(s * SEG, SEG)],
                             out_hbm.at[pl.ds(base + s * SEG, SEG)],
                             out_sem))
    for cp in out_cps:
        cp.wait()


def kernel(info_state, rank_w, suit_w, card_w):
    cards = info_state[:, 2].reshape(-1)
    return _embed(cards, rank_w, suit_w, card_w)


# speedup vs baseline: 1.0790x; 1.0790x over previous
import functools

import jax
import jax.numpy as jnp
from jax import lax
from jax.experimental import pallas as pl
from jax.experimental.pallas import tpu as pltpu
from jax.experimental.pallas import tpu_sc as plsc

NC = 2
NS = 16
L = 16
NW = NC * NS

BATCH = 16384
DIM = 128
ROWS_PER_W = BATCH // NW
NSEG = 2
SEG = ROWS_PER_W // NSEG

_MAGIC = 12582912.0


@functools.partial(
    pl.kernel,
    out_type=jax.ShapeDtypeStruct((BATCH, DIM), jnp.float32),
    mesh=plsc.VectorSubcoreMesh(core_axis_name="c", subcore_axis_name="s"),
    scratch_types=[
        pltpu.VMEM((3, DIM), jnp.float32),
        pltpu.VMEM((1, DIM), jnp.float32),
        pltpu.VMEM((52, DIM), jnp.float32),
        pltpu.VMEM((53, DIM), jnp.float32),
        pltpu.VMEM((ROWS_PER_W,), jnp.float32),
        pltpu.VMEM((ROWS_PER_W, DIM), jnp.float32),
        pltpu.SemaphoreType.DMA,
        pltpu.SemaphoreType.DMA,
        pltpu.SemaphoreType.DMA,
    ],
    compiler_params=pltpu.CompilerParams(
        needs_layout_passes=False, use_tc_tiling_on_sc=False
    ),
)
def _embed(cards_hbm, rank_hbm, suit_hbm, card_hbm, out_hbm,
           rank_v, suit_v, card_v, comb_v, cards_v, rows_v,
           sem, cards_sem, out_sem):
    wid = lax.axis_index("s") * NC + lax.axis_index("c")
    base = wid * ROWS_PER_W

    cards_cp = pltpu.async_copy(
        cards_hbm.at[pl.ds(base, ROWS_PER_W)], cards_v, cards_sem)
    loads = [
        pltpu.async_copy(rank_hbm, rank_v, sem),
        pltpu.async_copy(suit_hbm, suit_v, sem),
        pltpu.async_copy(card_hbm, card_v, sem),
    ]
    for cp in loads:
        cp.wait()

    for d in range(DIM // L):
        ds = pl.ds(d * L, L)
        suit_row = suit_v[0, ds]
        for j in range(3):
            rank_v[j, ds] = rank_v[j, ds] + suit_row
        comb_v[52, ds] = suit_row
    for k in range(52):
        rk = min(k // 4, 2)
        for d in range(DIM // L):
            ds = pl.ds(d * L, L)
            comb_v[k, ds] = card_v[k, ds] + rank_v[rk, ds]

    def row_loads(cj):
        return [comb_v[cj, pl.ds(d * L, L)] for d in range(DIM // L)]

    def row_stores(row, vals):
        for d in range(DIM // L):
            rows_v[row, pl.ds(d * L, L)] = vals[d]

    def expand(lo, hi):
        @plsc.parallel_loop(lo, hi)
        def _(g):
            cards_f = cards_v[pl.ds(g * L, L)]
            rounded = (cards_f + _MAGIC) - _MAGIC
            c = rounded.astype(jnp.int32)
            idx = jnp.where(c < 0, 52, jnp.minimum(c, 51))
            cjs = [idx[j] for j in range(L)]
            vals = row_loads(cjs[0])
            for j in range(1, L):
                nxt = []
                for d in range(DIM // L):
                    nxt.append(comb_v[cjs[j], pl.ds(d * L, L)])
                    rows_v[g * L + (j - 1), pl.ds(d * L, L)] = vals[d]
                vals = nxt
            row_stores(g * L + (L - 1), vals)

    cards_cp.wait()
    out_cps = []
    for s in range(NSEG):
        expand(s * (SEG // L), (s + 1) * (SEG // L))
        out_cps.append(
            pltpu.async_copy(rows_v.at[pl.ds(s * SEG, SEG)],
                             out_hbm.at[pl.ds(base + s * SEG, SEG)],
                             out_sem))
    for cp in out_cps:
        cp.wait()


def kernel(info_state, rank_w, suit_w, card_w):
    cards = info_state[:, 2].reshape(-1)
    return _embed(cards, rank_w, suit_w, card_w)
